# Initial kernel scaffold; baseline (speedup 1.0000x reference)
import jax
import jax.numpy as jnp
from jax.experimental import pallas as pl


def kernel(x, edge, W1, b1, W2, b2):
    raise NotImplementedError("write your pallas kernel here")


# speedup vs baseline: 8.9546x; 8.9546x over previous
import functools

import jax
import jax.numpy as jnp
from jax import lax
from jax.experimental import pallas as pl
from jax.experimental.pallas import tpu as pltpu
from jax.experimental.pallas import tpu_sc as plsc

N_NODES = 10000
N_DUMP = 112
N_ACC = N_NODES + N_DUMP
N_PAD = 10240
E_TOTAL = 320000
DIM1 = 128
DIM2 = 32

NT = 16
EPT = E_TOTAL // NT
CH = 128
NCH = 160
EPP = NCH * CH
RCH = 128
RPT = N_PAD // NT

_mesh = plsc.VectorSubcoreMesh(core_axis_name="c", subcore_axis_name="s")


def _mm_split(x, w):
    bn = 1000
    if x.ndim == 2:
        n, k = x.shape
        m = w.shape[1]
        hm = m // 2
        ws = w.reshape(k, 2, hm).transpose(1, 0, 2)

        def body(x_ref, w_ref, o_ref):
            o_ref[0] = jnp.dot(x_ref[...], w_ref[0],
                               preferred_element_type=jnp.float32)

        return pl.pallas_call(
            body,
            grid=(n // bn, 2),
            in_specs=[
                pl.BlockSpec((bn, k), lambda i, c: (i, 0)),
                pl.BlockSpec((1, k, hm), lambda i, c: (c, 0, 0)),
            ],
            out_specs=pl.BlockSpec((1, bn, hm), lambda i, c: (c, i, 0)),
            out_shape=jax.ShapeDtypeStruct((2, n, hm), jnp.float32),
        )(x, ws)

    _, n, hk = x.shape
    m = w.shape[1]
    hm = m // 2
    ws = w.reshape(2, hk, 2, hm).transpose(2, 0, 1, 3)

    def body(x_ref, w_ref, o_ref):
        xb = x_ref[...]
        wb = w_ref[0]
        o_ref[0] = (
            jnp.dot(xb[0], wb[0], preferred_element_type=jnp.float32)
            + jnp.dot(xb[1], wb[1], preferred_element_type=jnp.float32))

    return pl.pallas_call(
        body,
        grid=(n // bn, 2),
        in_specs=[
            pl.BlockSpec((2, bn, hk), lambda i, c: (0, i, 0)),
            pl.BlockSpec((1, 2, hk, hm), lambda i, c: (c, 0, 0, 0)),
        ],
        out_specs=pl.BlockSpec((1, bn, hm), lambda i, c: (c, i, 0)),
        out_shape=jax.ShapeDtypeStruct((2, n, hm), jnp.float32),
    )(x, ws)


def _make_sc_layer(width, with_hist):
    dim = 2 * width
    nq = width // 16

    out_type = [jax.ShapeDtypeStruct((2, N_ACC, width), jnp.float32)]
    if with_hist:
        out_type += [jax.ShapeDtypeStruct((N_PAD,), jnp.float32),
                     jax.ShapeDtypeStruct((N_PAD,), jnp.float32)]

    scratch = dict(
        tbl_n=pltpu.VMEM_SHARED((N_ACC, width), jnp.float32),
        acc_e=pltpu.VMEM_SHARED((N_ACC, width), jnp.float32),
        nb=pltpu.VMEM((CH,), jnp.int32),
        hb=pltpu.VMEM((CH,), jnp.int32),
        rows_a=pltpu.VMEM((CH, width), jnp.float32),
        rows_b=pltpu.VMEM((CH, width), jnp.float32),
        dinv_b=pltpu.VMEM((RPT,), jnp.float32),
        binv_b=pltpu.VMEM((RPT,), jnp.float32),
        bias_b=pltpu.VMEM((width,), jnp.float32),
    )
    if with_hist:
        scratch["hist_d"] = pltpu.VMEM_SHARED((N_PAD,), jnp.float32)
        scratch["hist_b"] = pltpu.VMEM_SHARED((N_PAD,), jnp.float32)
        scratch["ones_v"] = pltpu.VMEM((CH,), jnp.float32)

    def body(*refs, tbl_n, acc_e, nb, hb, rows_a, rows_b,
             dinv_b, binv_b, bias_b, **extra):
        if with_hist:
            (xl, edge5, bias, out_h, dinv_o, binv_o) = refs
            hist_d, hist_b, ones_v = (extra["hist_d"], extra["hist_b"],
                                      extra["ones_v"])
        else:
            (xl, edge5, bias, dinv_i, binv_i, out_h) = refs

        c = lax.axis_index("c")
        t = lax.axis_index("s")
        row0 = t * RPT
        col0 = c * width
        nrch = jnp.where(t == NT - 1, 4, 5)

        z16 = jnp.zeros((16,), jnp.float32)

        pltpu.sync_copy(bias.at[pl.ds(col0, width)], bias_b)

        def zrow(i, _):
            for q in range(nq):
                rows_b[i, pl.ds(16 * q, 16)] = z16
            return 0
        lax.fori_loop(0, CH, zrow, 0)

        def zacc(k, _):
            r = row0 + RCH * k
            pltpu.sync_copy(rows_b, acc_e.at[pl.ds(r, RCH)])
            return 0
        lax.fori_loop(0, nrch, zacc, 0)

        def stg(k, _):
            r = row0 + RCH * k
            pltpu.sync_copy(xl.at[c, pl.ds(r, RCH)], rows_a)
            pltpu.sync_copy(rows_a, tbl_n.at[pl.ds(r, RCH)])
            return 0
        lax.fori_loop(0, nrch, stg, 0)

        if with_hist:
            def zv(i, _):
                binv_b[pl.ds(16 * i, 16)] = z16
                return 0
            lax.fori_loop(0, RPT // 16, zv, 0)
            pltpu.sync_copy(binv_b, hist_d.at[pl.ds(row0, RPT)])
            pltpu.sync_copy(binv_b, hist_b.at[pl.ds(row0, RPT)])

            def ov(i, _):
                ones_v[pl.ds(16 * i, 16)] = jnp.ones((16,), jnp.float32)
                return 0
            lax.fori_loop(0, CH // 16, ov, 0)
        else:
            pltpu.sync_copy(dinv_i.at[pl.ds(row0, RPT)], dinv_b)
            pltpu.sync_copy(binv_i.at[pl.ds(row0, RPT)], binv_b)

        plsc.subcore_barrier()

        def prop(gsrc, sdst, gather_by_node, do_hist):
            def chunk(j, _):
                pltpu.sync_copy(edge5.at[0, t, j, 0], nb)
                pltpu.sync_copy(edge5.at[1, t, j, 0], hb)
                gi = nb if gather_by_node else hb
                si = hb if gather_by_node else nb
                pltpu.sync_copy(gsrc.at[gi], rows_a)
                pltpu.sync_copy(rows_a, sdst.at[si], add=True)
                if do_hist:
                    pltpu.sync_copy(ones_v, hist_d.at[nb], add=True)
                    pltpu.sync_copy(ones_v, hist_b.at[hb], add=True)
                return 0
            lax.fori_loop(0, NCH, chunk, 0)

        prop(tbl_n, acc_e, True, with_hist)
        plsc.subcore_barrier()

        if with_hist:
            pltpu.sync_copy(hist_d.at[pl.ds(row0, RPT)], dinv_b)
            pltpu.sync_copy(hist_b.at[pl.ds(row0, RPT)], binv_b)

            def inv(i, _):
                sl = pl.ds(16 * i, 16)
                v = dinv_b[sl]
                dinv_b[sl] = jnp.where(v > 0.0, 1.0 / v, 0.0)
                w = binv_b[sl]
                binv_b[sl] = jnp.where(w > 0.0, 1.0 / w, 0.0)
                return 0
            lax.fori_loop(0, RPT // 16, inv, 0)

            @pl.when(c == 0)
            def _():
                pltpu.sync_copy(dinv_b, dinv_o.at[pl.ds(row0, RPT)])
                pltpu.sync_copy(binv_b, binv_o.at[pl.ds(row0, RPT)])

        def scale_e(k, _):
            r = row0 + RCH * k
            pltpu.sync_copy(acc_e.at[pl.ds(r, RCH)], rows_a)

            def blk(m, _):
                bv = binv_b[pl.ds(RCH * k + 16 * m, 16)]
                for rr in range(16):
                    ridx = 16 * m + rr
                    sv = jnp.full((16,), bv[rr], jnp.float32)
                    for q in range(nq):
                        sl = pl.ds(16 * q, 16)
                        rows_a[ridx, sl] = rows_a[ridx, sl] * sv
                return 0
            lax.fori_loop(0, RCH // 16, blk, 0)
            pltpu.sync_copy(rows_a, acc_e.at[pl.ds(r, RCH)])
            return 0
        lax.fori_loop(0, nrch, scale_e, 0)

        def zrow2(i, _):
            for q in range(nq):
                rows_b[i, pl.ds(16 * q, 16)] = z16
            return 0
        lax.fori_loop(0, CH, zrow2, 0)

        def zacc2(k, _):
            r = row0 + RCH * k
            pltpu.sync_copy(rows_b, tbl_n.at[pl.ds(r, RCH)])
            return 0
        lax.fori_loop(0, nrch, zacc2, 0)

        plsc.subcore_barrier()

        prop(acc_e, tbl_n, False, False)
        plsc.subcore_barrier()

        def fin(k, _):
            r = row0 + RCH * k
            pltpu.sync_copy(tbl_n.at[pl.ds(r, RCH)], rows_a)

            def blk(m, _):
                dv = dinv_b[pl.ds(RCH * k + 16 * m, 16)]
                for rr in range(16):
                    ridx = 16 * m + rr
                    sv = jnp.full((16,), dv[rr], jnp.float32)
                    for q in range(nq):
                        sl = pl.ds(16 * q, 16)
                        v = rows_a[ridx, sl] * sv + bias_b[sl]
                        rows_a[ridx, sl] = jnp.maximum(v, 0.0)
                return 0
            lax.fori_loop(0, RCH // 16, blk, 0)
            pltpu.sync_copy(rows_a, out_h.at[c, pl.ds(r, RCH)])
            return 0
        lax.fori_loop(0, nrch, fin, 0)

    return pl.kernel(body, out_type=out_type, mesh=_mesh,
                     scratch_types=scratch,
                     compiler_params=pltpu.CompilerParams(
                         use_tc_tiling_on_sc=False))


_sc1 = _make_sc_layer(DIM1 // 2, with_hist=True)
_sc2 = _make_sc_layer(DIM2 // 2, with_hist=False)


def _pad_edges(edge):
    er = edge.reshape(2, NT, EPT)
    npad = EPP - EPT
    pad = (N_NODES
           + (jnp.arange(npad, dtype=jnp.int32)[None, :]
              + 7 * jnp.arange(NT, dtype=jnp.int32)[:, None]) % N_DUMP)
    pad = jnp.broadcast_to(pad[None], (2, NT, npad))
    return jnp.concatenate([er, pad], axis=2).reshape(2, NT, NCH, 1, CH)


@jax.jit
def kernel(x, edge, W1, b1, W2, b2):
    edge5 = _pad_edges(edge)
    xl1 = _mm_split(x, W1)
    xl1p = jnp.pad(xl1, ((0, 0), (0, N_DUMP), (0, 0)))
    h, dinv, binv = _sc1(xl1p, edge5, b1)
    xl2 = _mm_split(h[:, :N_NODES], W2)
    xl2p = jnp.pad(xl2, ((0, 0), (0, N_DUMP), (0, 0)))
    out_s = _sc2(xl2p, edge5, b2, dinv, binv)
    if isinstance(out_s, (list, tuple)):
        out_s = out_s[0]
    out_s = out_s[:, :N_NODES]
    return out_s.transpose(1, 0, 2).reshape(N_NODES, DIM2)


# speedup vs baseline: 13.6455x; 1.5239x over previous
import functools

import jax
import jax.numpy as jnp
from jax import lax
from jax.experimental import pallas as pl
from jax.experimental.pallas import tpu as pltpu
from jax.experimental.pallas import tpu_sc as plsc

N_NODES = 10000
N_DUMP = 112
N_ACC = N_NODES + N_DUMP
N_PAD = 10240
E_TOTAL = 320000
DIM1 = 128
DIM2 = 32

NT = 16
EPT = E_TOTAL // NT
CH = 128
NCH = 160
EPP = NCH * CH
RCH = 128
RPT = N_PAD // NT

_mesh = plsc.VectorSubcoreMesh(core_axis_name="c", subcore_axis_name="s")


def _mm_split(x, w):
    bn = 1000
    if x.ndim == 2:
        n, k = x.shape
        m = w.shape[1]
        hm = m // 2
        ws = w.reshape(k, 2, hm).transpose(1, 0, 2)

        def body(x_ref, w_ref, o_ref):
            o_ref[0] = jnp.dot(x_ref[...], w_ref[0],
                               preferred_element_type=jnp.float32)

        return pl.pallas_call(
            body,
            grid=(n // bn, 2),
            in_specs=[
                pl.BlockSpec((bn, k), lambda i, c: (i, 0)),
                pl.BlockSpec((1, k, hm), lambda i, c: (c, 0, 0)),
            ],
            out_specs=pl.BlockSpec((1, bn, hm), lambda i, c: (c, i, 0)),
            out_shape=jax.ShapeDtypeStruct((2, n, hm), jnp.float32),
        )(x, ws)

    _, n, hk = x.shape
    m = w.shape[1]
    hm = m // 2
    ws = w.reshape(2, hk, 2, hm).transpose(2, 0, 1, 3)

    def body(x_ref, w_ref, o_ref):
        xb = x_ref[...]
        wb = w_ref[0]
        o_ref[0] = (
            jnp.dot(xb[0], wb[0], preferred_element_type=jnp.float32)
            + jnp.dot(xb[1], wb[1], preferred_element_type=jnp.float32))

    return pl.pallas_call(
        body,
        grid=(n // bn, 2),
        in_specs=[
            pl.BlockSpec((2, bn, hk), lambda i, c: (0, i, 0)),
            pl.BlockSpec((1, 2, hk, hm), lambda i, c: (c, 0, 0, 0)),
        ],
        out_specs=pl.BlockSpec((1, bn, hm), lambda i, c: (c, i, 0)),
        out_shape=jax.ShapeDtypeStruct((2, n, hm), jnp.float32),
    )(x, ws)


def _make_sc_layer(width, with_hist):
    dim = 2 * width
    nq = width // 16

    out_type = [jax.ShapeDtypeStruct((2, N_ACC, width), jnp.float32)]
    if with_hist:
        out_type += [jax.ShapeDtypeStruct((N_PAD,), jnp.float32),
                     jax.ShapeDtypeStruct((N_PAD,), jnp.float32)]

    scratch = dict(
        tbl_n=pltpu.VMEM_SHARED((N_ACC, width), jnp.float32),
        acc_e=pltpu.VMEM_SHARED((N_ACC, width), jnp.float32),
        nb=pltpu.VMEM((CH,), jnp.int32),
        hb=pltpu.VMEM((CH,), jnp.int32),
        nb2=pltpu.VMEM((CH,), jnp.int32),
        hb2=pltpu.VMEM((CH,), jnp.int32),
        sem_a=pltpu.SemaphoreType.DMA,
        sem_b=pltpu.SemaphoreType.DMA,
        sem_ia=pltpu.SemaphoreType.DMA,
        sem_ib=pltpu.SemaphoreType.DMA,
        sem_h=pltpu.SemaphoreType.DMA,
        rows_a=pltpu.VMEM((CH, width), jnp.float32),
        rows_b=pltpu.VMEM((CH, width), jnp.float32),
        dinv_b=pltpu.VMEM((RPT,), jnp.float32),
        binv_b=pltpu.VMEM((RPT,), jnp.float32),
        bias_b=pltpu.VMEM((width,), jnp.float32),
    )
    if with_hist:
        scratch["hist_d"] = pltpu.VMEM_SHARED((N_PAD,), jnp.float32)
        scratch["hist_b"] = pltpu.VMEM_SHARED((N_PAD,), jnp.float32)
        scratch["ones_v"] = pltpu.VMEM((CH,), jnp.float32)

    def body(*refs, tbl_n, acc_e, nb, hb, nb2, hb2,
             sem_a, sem_b, sem_ia, sem_ib, sem_h, rows_a, rows_b,
             dinv_b, binv_b, bias_b, **extra):
        if with_hist:
            (xl, edge5, bias, out_h, dinv_o, binv_o) = refs
            hist_d, hist_b, ones_v = (extra["hist_d"], extra["hist_b"],
                                      extra["ones_v"])
        else:
            (xl, edge5, bias, dinv_i, binv_i, out_h) = refs

        c = lax.axis_index("c")
        t = lax.axis_index("s")
        row0 = t * RPT
        col0 = c * width
        nrch = jnp.where(t == NT - 1, 4, 5)

        z16 = jnp.zeros((16,), jnp.float32)

        pltpu.sync_copy(bias.at[pl.ds(col0, width)], bias_b)

        def zrow(i, _):
            for q in range(nq):
                rows_b[i, pl.ds(16 * q, 16)] = z16
            return 0
        lax.fori_loop(0, CH, zrow, 0)

        def zacc(k, _):
            r = row0 + RCH * k
            pltpu.sync_copy(rows_b, acc_e.at[pl.ds(r, RCH)])
            return 0
        lax.fori_loop(0, nrch, zacc, 0)

        def stg(k, _):
            r = row0 + RCH * k
            pltpu.sync_copy(xl.at[c, pl.ds(r, RCH)], rows_a)
            pltpu.sync_copy(rows_a, tbl_n.at[pl.ds(r, RCH)])
            return 0
        lax.fori_loop(0, nrch, stg, 0)

        if with_hist:
            def zv(i, _):
                binv_b[pl.ds(16 * i, 16)] = z16
                return 0
            lax.fori_loop(0, RPT // 16, zv, 0)
            pltpu.sync_copy(binv_b, hist_d.at[pl.ds(row0, RPT)])
            pltpu.sync_copy(binv_b, hist_b.at[pl.ds(row0, RPT)])

            def ov(i, _):
                ones_v[pl.ds(16 * i, 16)] = jnp.ones((16,), jnp.float32)
                return 0
            lax.fori_loop(0, CH // 16, ov, 0)
        else:
            pltpu.sync_copy(dinv_i.at[pl.ds(row0, RPT)], dinv_b)
            pltpu.sync_copy(binv_i.at[pl.ds(row0, RPT)], binv_b)

        plsc.subcore_barrier()

        def prop(gsrc, sdst, gather_by_node, do_hist):
            NPAIR = NCH // 2

            def idx_fetch(j, nref, href, sem):
                da = pltpu.async_copy(edge5.at[0, t, j, 0], nref, sem)
                db = pltpu.async_copy(edge5.at[1, t, j, 0], href, sem)
                return da, db

            def proc(j, nref, href, rows, gsem, other_rows):
                gi = nref if gather_by_node else href
                si = href if gather_by_node else nref
                pltpu.make_async_copy(gsrc.at[gi], rows, gsem).wait()
                hd = []
                if do_hist:
                    hd.append(pltpu.async_copy(ones_v, hist_d.at[nref],
                                               sem_h, add=True))
                    hd.append(pltpu.async_copy(ones_v, hist_b.at[href],
                                               sem_h, add=True))
                pltpu.sync_copy(rows, sdst.at[si], add=True)
                for d in hd:
                    d.wait()

            def pair(g, _):
                j0 = 2 * g
                pltpu.make_async_copy(edge5.at[0, t, j0 + 1, 0], nb2,
                                      sem_ib).wait()
                pltpu.make_async_copy(edge5.at[1, t, j0 + 1, 0], hb2,
                                      sem_ib).wait()
                gi2 = nb2 if gather_by_node else hb2
                pltpu.async_copy(gsrc.at[gi2], rows_b, sem_b)
                proc(j0, nb, hb, rows_a, sem_a, rows_b)

                @pl.when(g < NPAIR - 1)
                def _():
                    da, db = idx_fetch(j0 + 2, nb, hb, sem_ia)
                    da.wait()
                    db.wait()
                    gi0 = nb if gather_by_node else hb
                    pltpu.async_copy(gsrc.at[gi0], rows_a, sem_a)

                proc(j0 + 1, nb2, hb2, rows_b, sem_b, rows_a)

                @pl.when(g < NPAIR - 1)
                def _():
                    idx_fetch(j0 + 3, nb2, hb2, sem_ib)
                return 0

            da, db = idx_fetch(0, nb, hb, sem_ia)
            da.wait()
            db.wait()
            gi0 = nb if gather_by_node else hb
            pltpu.async_copy(gsrc.at[gi0], rows_a, sem_a)
            idx_fetch(1, nb2, hb2, sem_ib)
            lax.fori_loop(0, NPAIR, pair, 0)

        prop(tbl_n, acc_e, True, with_hist)
        plsc.subcore_barrier()

        if with_hist:
            pltpu.sync_copy(hist_d.at[pl.ds(row0, RPT)], dinv_b)
            pltpu.sync_copy(hist_b.at[pl.ds(row0, RPT)], binv_b)

            def inv(i, _):
                sl = pl.ds(16 * i, 16)
                v = dinv_b[sl]
                dinv_b[sl] = jnp.where(v > 0.0, 1.0 / v, 0.0)
                w = binv_b[sl]
                binv_b[sl] = jnp.where(w > 0.0, 1.0 / w, 0.0)
                return 0
            lax.fori_loop(0, RPT // 16, inv, 0)

            @pl.when(c == 0)
            def _():
                pltpu.sync_copy(dinv_b, dinv_o.at[pl.ds(row0, RPT)])
                pltpu.sync_copy(binv_b, binv_o.at[pl.ds(row0, RPT)])

        def scale_e(k, _):
            r = row0 + RCH * k
            pltpu.sync_copy(acc_e.at[pl.ds(r, RCH)], rows_a)

            def blk(m, _):
                bv = binv_b[pl.ds(RCH * k + 16 * m, 16)]
                for rr in range(16):
                    ridx = 16 * m + rr
                    sv = jnp.full((16,), bv[rr], jnp.float32)
                    for q in range(nq):
                        sl = pl.ds(16 * q, 16)
                        rows_a[ridx, sl] = rows_a[ridx, sl] * sv
                return 0
            lax.fori_loop(0, RCH // 16, blk, 0)
            pltpu.sync_copy(rows_a, acc_e.at[pl.ds(r, RCH)])
            return 0
        lax.fori_loop(0, nrch, scale_e, 0)

        def zrow2(i, _):
            for q in range(nq):
                rows_b[i, pl.ds(16 * q, 16)] = z16
            return 0
        lax.fori_loop(0, CH, zrow2, 0)

        def zacc2(k, _):
            r = row0 + RCH * k
            pltpu.sync_copy(rows_b, tbl_n.at[pl.ds(r, RCH)])
            return 0
        lax.fori_loop(0, nrch, zacc2, 0)

        plsc.subcore_barrier()

        prop(acc_e, tbl_n, False, False)
        plsc.subcore_barrier()

        def fin(k, _):
            r = row0 + RCH * k
            pltpu.sync_copy(tbl_n.at[pl.ds(r, RCH)], rows_a)

            def blk(m, _):
                dv = dinv_b[pl.ds(RCH * k + 16 * m, 16)]
                for rr in range(16):
                    ridx = 16 * m + rr
                    sv = jnp.full((16,), dv[rr], jnp.float32)
                    for q in range(nq):
                        sl = pl.ds(16 * q, 16)
                        v = rows_a[ridx, sl] * sv + bias_b[sl]
                        rows_a[ridx, sl] = jnp.maximum(v, 0.0)
                return 0
            lax.fori_loop(0, RCH // 16, blk, 0)
            pltpu.sync_copy(rows_a, out_h.at[c, pl.ds(r, RCH)])
            return 0
        lax.fori_loop(0, nrch, fin, 0)

    return pl.kernel(body, out_type=out_type, mesh=_mesh,
                     scratch_types=scratch,
                     compiler_params=pltpu.CompilerParams(
                         use_tc_tiling_on_sc=False))


_sc1 = _make_sc_layer(DIM1 // 2, with_hist=True)
_sc2 = _make_sc_layer(DIM2 // 2, with_hist=False)


def _pad_edges(edge):
    er = edge.reshape(2, NT, EPT)
    npad = EPP - EPT
    pad = (N_NODES
           + (jnp.arange(npad, dtype=jnp.int32)[None, :]
              + 7 * jnp.arange(NT, dtype=jnp.int32)[:, None]) % N_DUMP)
    pad = jnp.broadcast_to(pad[None], (2, NT, npad))
    return jnp.concatenate([er, pad], axis=2).reshape(2, NT, NCH, 1, CH)


@jax.jit
def kernel(x, edge, W1, b1, W2, b2):
    edge5 = _pad_edges(edge)
    xl1 = _mm_split(x, W1)
    xl1p = jnp.pad(xl1, ((0, 0), (0, N_DUMP), (0, 0)))
    h, dinv, binv = _sc1(xl1p, edge5, b1)
    xl2 = _mm_split(h[:, :N_NODES], W2)
    xl2p = jnp.pad(xl2, ((0, 0), (0, N_DUMP), (0, 0)))
    out_s = _sc2(xl2p, edge5, b2, dinv, binv)
    if isinstance(out_s, (list, tuple)):
        out_s = out_s[0]
    out_s = out_s[:, :N_NODES]
    return out_s.transpose(1, 0, 2).reshape(N_NODES, DIM2)
